# baseline (device time: 198466 ns/iter reference)
import jax
import jax.numpy as jnp
from jax import lax
from jax.experimental import pallas as pl
from jax.experimental.pallas import tpu as pltpu


def kernel(x, pi):
    def body(pi_ref, x_ref, out_ref, send_sem, recv_sem):
        my_x = lax.axis_index("x")
        my_y = lax.axis_index("y")
        my_z = lax.axis_index("z")
        dest_z = pi_ref[my_z]

        rdma = pltpu.make_async_remote_copy(
            src_ref=x_ref,
            dst_ref=out_ref,
            send_sem=send_sem,
            recv_sem=recv_sem,
            device_id=(my_x, my_y, dest_z),
            device_id_type=pl.DeviceIdType.MESH,
        )
        rdma.start()
        rdma.wait()

    return pl.pallas_call(
        body,
        out_shape=jax.ShapeDtypeStruct(x.shape, jnp.float32),
        in_specs=[
            pl.BlockSpec(memory_space=pltpu.SMEM),
            pl.BlockSpec(memory_space=pltpu.VMEM),
        ],
        out_specs=pl.BlockSpec(memory_space=pltpu.VMEM),
        scratch_shapes=[
            pltpu.SemaphoreType.DMA,
            pltpu.SemaphoreType.DMA,
        ],
        compiler_params=pltpu.CompilerParams(has_side_effects=True),
    )(pi, x)


# device time: 113063 ns/iter; 1.7554x vs baseline; 1.7554x over previous
import jax
import jax.numpy as jnp
from jax import lax
from jax.experimental import pallas as pl
from jax.experimental.pallas import tpu as pltpu

Z = 4


def kernel(x, pi):
    _, m, n = x.shape
    mq = m // Z

    def body(pi_ref, x_ref, out_ref,
             a_send, a_recv, b1x_send, b1x_recv, b1y_send, b1y_recv,
             b2_send, b2_recv):
        my_x = lax.axis_index("x")
        my_y = lax.axis_index("y")
        my_z = lax.axis_index("z")
        dest_z = pi_ref[my_z]

        q = 2 * my_x + my_y
        q_x = 2 * (1 - my_x) + my_y
        q_y = 2 * my_x + (1 - my_y)

        def rows(qi):
            return pl.ds(qi * mq, mq)

        a = pltpu.make_async_remote_copy(
            src_ref=x_ref.at[:, rows(q), :],
            dst_ref=out_ref.at[:, rows(q), :],
            send_sem=a_send,
            recv_sem=a_recv,
            device_id=(my_x, my_y, dest_z),
            device_id_type=pl.DeviceIdType.MESH,
        )
        a.start()
        a.wait()

        b1x = pltpu.make_async_remote_copy(
            src_ref=out_ref.at[:, rows(q), :],
            dst_ref=out_ref.at[:, rows(q), :],
            send_sem=b1x_send,
            recv_sem=b1x_recv,
            device_id=(1 - my_x, my_y, my_z),
            device_id_type=pl.DeviceIdType.MESH,
        )
        b1y = pltpu.make_async_remote_copy(
            src_ref=out_ref.at[:, rows(q), :],
            dst_ref=out_ref.at[:, rows(q), :],
            send_sem=b1y_send,
            recv_sem=b1y_recv,
            device_id=(my_x, 1 - my_y, my_z),
            device_id_type=pl.DeviceIdType.MESH,
        )
        b1x.start()
        b1y.start()
        b1x.wait()
        b1y.wait()

        b2 = pltpu.make_async_remote_copy(
            src_ref=out_ref.at[:, rows(q_y), :],
            dst_ref=out_ref.at[:, rows(q_y), :],
            send_sem=b2_send,
            recv_sem=b2_recv,
            device_id=(1 - my_x, my_y, my_z),
            device_id_type=pl.DeviceIdType.MESH,
        )
        b2.start()
        b2.wait()

    return pl.pallas_call(
        body,
        out_shape=jax.ShapeDtypeStruct(x.shape, jnp.float32),
        in_specs=[
            pl.BlockSpec(memory_space=pltpu.SMEM),
            pl.BlockSpec(memory_space=pltpu.VMEM),
        ],
        out_specs=pl.BlockSpec(memory_space=pltpu.VMEM),
        scratch_shapes=[pltpu.SemaphoreType.DMA] * 8,
        compiler_params=pltpu.CompilerParams(has_side_effects=True),
    )(pi, x)


# device time: 83067 ns/iter; 2.3892x vs baseline; 1.3611x over previous
import jax
import jax.numpy as jnp
from jax import lax
from jax.experimental import pallas as pl
from jax.experimental.pallas import tpu as pltpu

Z = 4
NCHUNK = 4


def kernel(x, pi):
    _, m, n = x.shape
    mq = m // Z
    mc = mq // NCHUNK

    def body(pi_ref, x_ref, out_ref,
             a_send, a_recv, b1x_send, b1x_recv, b1y_send, b1y_recv,
             b2_send, b2_recv):
        my_x = lax.axis_index("x")
        my_y = lax.axis_index("y")
        my_z = lax.axis_index("z")
        dest_z = pi_ref[my_z]

        q = 2 * my_x + my_y
        q_x = 2 * (1 - my_x) + my_y
        q_y = 2 * my_x + (1 - my_y)

        def rows(qi, j):
            return pl.ds(qi * mq + j * mc, mc)

        def rdma(qi, j, dev, send_sems, recv_sems, src=None):
            return pltpu.make_async_remote_copy(
                src_ref=(src if src is not None
                         else out_ref).at[:, rows(qi, j), :],
                dst_ref=out_ref.at[:, rows(qi, j), :],
                send_sem=send_sems.at[j],
                recv_sem=recv_sems.at[j],
                device_id=dev,
                device_id_type=pl.DeviceIdType.MESH,
            )

        xnbr = (1 - my_x, my_y, my_z)
        ynbr = (my_x, 1 - my_y, my_z)

        a = [rdma(q, j, (my_x, my_y, dest_z), a_send, a_recv, src=x_ref)
             for j in range(NCHUNK)]
        for j in range(NCHUNK):
            a[j].start()

        b1x = [rdma(q, j, xnbr, b1x_send, b1x_recv) for j in range(NCHUNK)]
        b1y = [rdma(q, j, ynbr, b1y_send, b1y_recv) for j in range(NCHUNK)]
        for j in range(NCHUNK):
            a[j].wait_recv()
            b1x[j].start()
            b1y[j].start()

        b2 = [rdma(q_y, j, xnbr, b2_send, b2_recv) if j % 2 == 0
              else rdma(q_x, j, ynbr, b2_send, b2_recv)
              for j in range(NCHUNK)]
        for j in range(NCHUNK):
            if j % 2 == 0:
                b1y[j].wait_recv()
            else:
                b1x[j].wait_recv()
            b2[j].start()

        for j in range(NCHUNK):
            if j % 2 == 0:
                b1x[j].wait_recv()
            else:
                b1y[j].wait_recv()
            b2[j].wait_recv()
        for j in range(NCHUNK):
            a[j].wait_send()
            b1x[j].wait_send()
            b1y[j].wait_send()
            b2[j].wait_send()

    sem = pltpu.SemaphoreType.DMA((NCHUNK,))
    return pl.pallas_call(
        body,
        out_shape=jax.ShapeDtypeStruct(x.shape, jnp.float32),
        in_specs=[
            pl.BlockSpec(memory_space=pltpu.SMEM),
            pl.BlockSpec(memory_space=pltpu.VMEM),
        ],
        out_specs=pl.BlockSpec(memory_space=pltpu.VMEM),
        scratch_shapes=[sem] * 8,
        compiler_params=pltpu.CompilerParams(has_side_effects=True),
    )(pi, x)


# device time: 75123 ns/iter; 2.6419x vs baseline; 1.1057x over previous
import jax
import jax.numpy as jnp
from jax import lax
from jax.experimental import pallas as pl
from jax.experimental.pallas import tpu as pltpu

Z = 4
NCHUNK = 8


def kernel(x, pi):
    _, m, n = x.shape
    mq = m // Z
    mc = mq // NCHUNK

    def body(pi_ref, x_ref, out_ref,
             a_send, a_recv, b1x_send, b1x_recv, b1y_send, b1y_recv,
             b2_send, b2_recv):
        my_x = lax.axis_index("x")
        my_y = lax.axis_index("y")
        my_z = lax.axis_index("z")
        dest_z = pi_ref[my_z]

        q = 2 * my_x + my_y
        q_x = 2 * (1 - my_x) + my_y
        q_y = 2 * my_x + (1 - my_y)

        def rows(qi, j):
            return pl.ds(qi * mq + j * mc, mc)

        def rdma(qi, j, dev, send_sems, recv_sems, src=None):
            return pltpu.make_async_remote_copy(
                src_ref=(src if src is not None
                         else out_ref).at[:, rows(qi, j), :],
                dst_ref=out_ref.at[:, rows(qi, j), :],
                send_sem=send_sems.at[j],
                recv_sem=recv_sems.at[j],
                device_id=dev,
                device_id_type=pl.DeviceIdType.MESH,
            )

        xnbr = (1 - my_x, my_y, my_z)
        ynbr = (my_x, 1 - my_y, my_z)

        src_z = jnp.int32(0)
        for i in range(Z):
            src_z = jnp.where(pi_ref[i] == my_z, jnp.int32(i), src_z)

        barrier_sem = pltpu.get_barrier_semaphore()
        for dev in [(my_x, my_y, src_z), xnbr, ynbr]:
            pl.semaphore_signal(
                barrier_sem, inc=1,
                device_id=dev, device_id_type=pl.DeviceIdType.MESH,
            )
        pl.semaphore_wait(barrier_sem, 3)

        a = [rdma(q, j, (my_x, my_y, dest_z), a_send, a_recv, src=x_ref)
             for j in range(NCHUNK)]
        for j in range(NCHUNK):
            a[j].start()

        b1x = [rdma(q, j, xnbr, b1x_send, b1x_recv) for j in range(NCHUNK)]
        b1y = [rdma(q, j, ynbr, b1y_send, b1y_recv) for j in range(NCHUNK)]
        for j in range(NCHUNK):
            a[j].wait_recv()
            b1x[j].start()
            b1y[j].start()

        b2 = [rdma(q_y, j, xnbr, b2_send, b2_recv) if j % 2 == 0
              else rdma(q_x, j, ynbr, b2_send, b2_recv)
              for j in range(NCHUNK)]
        for j in range(NCHUNK):
            if j % 2 == 0:
                b1y[j].wait_recv()
            else:
                b1x[j].wait_recv()
            b2[j].start()

        for j in range(NCHUNK):
            if j % 2 == 0:
                b1x[j].wait_recv()
            else:
                b1y[j].wait_recv()
            b2[j].wait_recv()
        for j in range(NCHUNK):
            a[j].wait_send()
            b1x[j].wait_send()
            b1y[j].wait_send()
            b2[j].wait_send()

    sem = pltpu.SemaphoreType.DMA((NCHUNK,))
    return pl.pallas_call(
        body,
        out_shape=jax.ShapeDtypeStruct(x.shape, jnp.float32),
        in_specs=[
            pl.BlockSpec(memory_space=pltpu.SMEM),
            pl.BlockSpec(memory_space=pltpu.VMEM),
        ],
        out_specs=pl.BlockSpec(memory_space=pltpu.VMEM),
        scratch_shapes=[sem] * 8,
        compiler_params=pltpu.CompilerParams(
            has_side_effects=True, collective_id=0
        ),
    )(pi, x)


# device time: 74069 ns/iter; 2.6795x vs baseline; 1.0142x over previous
import jax
import jax.numpy as jnp
from jax import lax
from jax.experimental import pallas as pl
from jax.experimental.pallas import tpu as pltpu

Z = 4
NCHUNK = 8


def kernel(x, pi):
    _, m, n = x.shape
    mq = m // Z
    mc = mq // NCHUNK

    def body(pi_ref, x_ref, out_ref,
             a_send, a_recv, b1x_send, b1x_recv, b1y_send, b1y_recv,
             b2_send, b2_recv, plane_sem):
        my_x = lax.axis_index("x")
        my_y = lax.axis_index("y")
        my_z = lax.axis_index("z")
        dest_z = pi_ref[my_z]

        q = 2 * my_x + my_y
        q_x = 2 * (1 - my_x) + my_y
        q_y = 2 * my_x + (1 - my_y)

        def rows(qi, j):
            return pl.ds(qi * mq + j * mc, mc)

        def rdma(qi, j, dev, send_sems, recv_sems, src=None):
            return pltpu.make_async_remote_copy(
                src_ref=(src if src is not None
                         else out_ref).at[:, rows(qi, j), :],
                dst_ref=out_ref.at[:, rows(qi, j), :],
                send_sem=send_sems.at[j],
                recv_sem=recv_sems.at[j],
                device_id=dev,
                device_id_type=pl.DeviceIdType.MESH,
            )

        xnbr = (1 - my_x, my_y, my_z)
        ynbr = (my_x, 1 - my_y, my_z)

        src_z = jnp.int32(0)
        for i in range(Z):
            src_z = jnp.where(pi_ref[i] == my_z, jnp.int32(i), src_z)

        barrier_sem = pltpu.get_barrier_semaphore()
        pl.semaphore_signal(
            barrier_sem, inc=1,
            device_id=(my_x, my_y, src_z), device_id_type=pl.DeviceIdType.MESH,
        )
        for dev in [xnbr, ynbr]:
            pl.semaphore_signal(
                plane_sem, inc=1,
                device_id=dev, device_id_type=pl.DeviceIdType.MESH,
            )
        pl.semaphore_wait(barrier_sem, 1)

        a = [rdma(q, j, (my_x, my_y, dest_z), a_send, a_recv, src=x_ref)
             for j in range(NCHUNK)]
        for j in range(NCHUNK):
            a[j].start()

        pl.semaphore_wait(plane_sem, 2)

        b1x = [rdma(q, j, xnbr, b1x_send, b1x_recv) for j in range(NCHUNK)]
        b1y = [rdma(q, j, ynbr, b1y_send, b1y_recv) for j in range(NCHUNK)]
        for j in range(NCHUNK):
            a[j].wait_recv()
            b1x[j].start()
            b1y[j].start()

        b2 = [rdma(q_y, j, xnbr, b2_send, b2_recv) if j % 2 == 0
              else rdma(q_x, j, ynbr, b2_send, b2_recv)
              for j in range(NCHUNK)]
        for j in range(NCHUNK):
            if j % 2 == 0:
                b1y[j].wait_recv()
            else:
                b1x[j].wait_recv()
            b2[j].start()

        for j in range(NCHUNK):
            if j % 2 == 0:
                b1x[j].wait_recv()
            else:
                b1y[j].wait_recv()
            b2[j].wait_recv()
        for j in range(NCHUNK):
            a[j].wait_send()
            b1x[j].wait_send()
            b1y[j].wait_send()
            b2[j].wait_send()

    sem = pltpu.SemaphoreType.DMA((NCHUNK,))
    return pl.pallas_call(
        body,
        out_shape=jax.ShapeDtypeStruct(x.shape, jnp.float32),
        in_specs=[
            pl.BlockSpec(memory_space=pltpu.SMEM),
            pl.BlockSpec(memory_space=pltpu.VMEM),
        ],
        out_specs=pl.BlockSpec(memory_space=pltpu.VMEM),
        scratch_shapes=[sem] * 8 + [pltpu.SemaphoreType.REGULAR],
        compiler_params=pltpu.CompilerParams(
            has_side_effects=True, collective_id=0
        ),
    )(pi, x)


# device time: 73344 ns/iter; 2.7060x vs baseline; 1.0099x over previous
import jax
import jax.numpy as jnp
from jax import lax
from jax.experimental import pallas as pl
from jax.experimental.pallas import tpu as pltpu

Z = 4
NCHUNK = 16
LAG = 3


def kernel(x, pi):
    _, m, n = x.shape
    mq = m // Z
    mc = mq // NCHUNK

    def body(pi_ref, x_ref, out_ref,
             a_send, a_recv, b1x_send, b1x_recv, b1y_send, b1y_recv,
             b2_send, b2_recv, plane_sem):
        my_x = lax.axis_index("x")
        my_y = lax.axis_index("y")
        my_z = lax.axis_index("z")
        dest_z = pi_ref[my_z]

        q = 2 * my_x + my_y
        q_x = 2 * (1 - my_x) + my_y
        q_y = 2 * my_x + (1 - my_y)

        def rows(qi, j):
            return pl.ds(qi * mq + j * mc, mc)

        def rdma(qi, j, dev, send_sems, recv_sems, src=None):
            return pltpu.make_async_remote_copy(
                src_ref=(src if src is not None
                         else out_ref).at[:, rows(qi, j), :],
                dst_ref=out_ref.at[:, rows(qi, j), :],
                send_sem=send_sems.at[j],
                recv_sem=recv_sems.at[j],
                device_id=dev,
                device_id_type=pl.DeviceIdType.MESH,
            )

        xnbr = (1 - my_x, my_y, my_z)
        ynbr = (my_x, 1 - my_y, my_z)

        src_z = jnp.int32(0)
        for i in range(Z):
            src_z = jnp.where(pi_ref[i] == my_z, jnp.int32(i), src_z)

        barrier_sem = pltpu.get_barrier_semaphore()
        pl.semaphore_signal(
            barrier_sem, inc=1,
            device_id=(my_x, my_y, src_z), device_id_type=pl.DeviceIdType.MESH,
        )
        for dev in [xnbr, ynbr]:
            pl.semaphore_signal(
                plane_sem, inc=1,
                device_id=dev, device_id_type=pl.DeviceIdType.MESH,
            )
        pl.semaphore_wait(barrier_sem, 1)

        a = [rdma(q, j, (my_x, my_y, dest_z), a_send, a_recv, src=x_ref)
             for j in range(NCHUNK)]
        for j in range(NCHUNK):
            a[j].start()

        pl.semaphore_wait(plane_sem, 2)

        b1x = [rdma(q, j, xnbr, b1x_send, b1x_recv) for j in range(NCHUNK)]
        b1y = [rdma(q, j, ynbr, b1y_send, b1y_recv) for j in range(NCHUNK)]
        b2 = [rdma(q_y, j, xnbr, b2_send, b2_recv) if j % 2 == 0
              else rdma(q_x, j, ynbr, b2_send, b2_recv)
              for j in range(NCHUNK)]

        def start_b2(k):
            if k % 2 == 0:
                b1y[k].wait_recv()
            else:
                b1x[k].wait_recv()
            b2[k].start()

        for j in range(NCHUNK):
            a[j].wait_recv()
            b1x[j].start()
            b1y[j].start()
        for k in range(NCHUNK):
            start_b2(k)

        for j in range(NCHUNK):
            if j % 2 == 0:
                b1x[j].wait_recv()
            else:
                b1y[j].wait_recv()
            b2[j].wait_recv()
        for j in range(NCHUNK):
            a[j].wait_send()
            b1x[j].wait_send()
            b1y[j].wait_send()
            b2[j].wait_send()

    sem = pltpu.SemaphoreType.DMA((NCHUNK,))
    return pl.pallas_call(
        body,
        out_shape=jax.ShapeDtypeStruct(x.shape, jnp.float32),
        in_specs=[
            pl.BlockSpec(memory_space=pltpu.SMEM),
            pl.BlockSpec(memory_space=pltpu.VMEM),
        ],
        out_specs=pl.BlockSpec(memory_space=pltpu.VMEM),
        scratch_shapes=[sem] * 8 + [pltpu.SemaphoreType.REGULAR],
        compiler_params=pltpu.CompilerParams(
            has_side_effects=True, collective_id=0
        ),
    )(pi, x)
